# baseline (speedup 1.0000x reference)
import functools

import jax
import jax.numpy as jnp
from jax import lax
from jax.experimental import pallas as pl
from jax.experimental.pallas import tpu as pltpu
from jax.experimental.pallas import tpu_sc as plsc

_N, _D, _H, _E, _TOPK = 32768, 768, 128, 64, 8
_BN = 1024
_SIZES = (8192, 8192, 8192, 8192)

_NC, _NS, _L = 2, 16, 16
_NW = _NC * _NS


def _logits_body_first(x_ref, w1_ref, b1_ref, w2_ref, b2_ref,
                       logits_ref, logits_t_ref):
    h = jnp.dot(x_ref[...], w1_ref[...], preferred_element_type=jnp.float32)
    h = h + b1_ref[...]
    h = h * jax.nn.sigmoid(h)
    w2 = w2_ref[...]
    b2 = b2_ref[...]
    logits = jnp.dot(h, w2, preferred_element_type=jnp.float32)
    logits_ref[...] = logits + b2
    lt = lax.dot_general(w2, h, (((0,), (1,)), ((), ())),
                         preferred_element_type=jnp.float32)
    logits_t_ref[...] = lt + b2.reshape(_E, 1)


def _logits_body_rest(prev_ref, x_ref, w1_ref, b1_ref, w2_ref, b2_ref,
                      logits_ref, logits_t_ref):
    del prev_ref
    _logits_body_first(x_ref, w1_ref, b1_ref, w2_ref, b2_ref,
                       logits_ref, logits_t_ref)


def _tc_logits_chunk(block0, nchunk, x, W1, b1r, W2, b2r, logits_prev):
    gc = nchunk // _BN
    x_spec = pl.BlockSpec((_BN, _D), lambda i: (block0 + i, 0))
    w_specs = [
        pl.BlockSpec((_D, _H), lambda i: (0, 0)),
        pl.BlockSpec((1, _H), lambda i: (0, 0)),
        pl.BlockSpec((_H, _E), lambda i: (0, 0)),
        pl.BlockSpec((1, _E), lambda i: (0, 0)),
    ]
    out_specs = [
        pl.BlockSpec((_BN, _E), lambda i: (block0 + i, 0)),
        pl.BlockSpec((_E, _BN), lambda i: (0, i)),
    ]
    out_shape = [
        jax.ShapeDtypeStruct((_N, _E), jnp.float32),
        jax.ShapeDtypeStruct((_E, nchunk), jnp.float32),
    ]
    if logits_prev is None:
        return pl.pallas_call(
            _logits_body_first,
            grid=(gc,),
            in_specs=[x_spec] + w_specs,
            out_specs=out_specs,
            out_shape=out_shape,
        )(x, W1, b1r, W2, b2r)
    prev_spec = pl.BlockSpec((_BN, _E), lambda i: (block0 + i, 0))
    return pl.pallas_call(
        _logits_body_rest,
        grid=(gc,),
        in_specs=[prev_spec, x_spec] + w_specs,
        out_specs=out_specs,
        out_shape=out_shape,
        input_output_aliases={0: 0},
    )(logits_prev, x, W1, b1r, W2, b2r)


def _make_sc_body(rw):
    def _sc_topk_body(lt_hbm, kl_hbm, kp_hbm, ki_hbm, lg_v, kl_v, kp_v, ki_v):
        wid = lax.axis_index("s") * _NC + lax.axis_index("c")
        base = wid * rw
        pltpu.sync_copy(lt_hbm.at[:, pl.ds(base, rw)], lg_v)

        neg_inf = jnp.full((_L,), -jnp.inf, jnp.float32)
        zero_i = jnp.zeros((_L,), jnp.int32)

        def one_group(t0):
            vs = [neg_inf] * _TOPK
            ix = [zero_i] * _TOPK
            for e in range(_E):
                nv = lg_v[e, pl.ds(t0, _L)]
                ne = jnp.full((_L,), e, jnp.int32)
                cs = [nv > vs[j] for j in range(_TOPK)]
                nvs, nis = [], []
                for j in range(_TOPK):
                    if j == 0:
                        nvs.append(jnp.where(cs[0], nv, vs[0]))
                        nis.append(jnp.where(cs[0], ne, ix[0]))
                    else:
                        innerv = jnp.where(cs[j - 1], vs[j - 1], nv)
                        inneri = jnp.where(cs[j - 1], ix[j - 1], ne)
                        nvs.append(jnp.where(cs[j], innerv, vs[j]))
                        nis.append(jnp.where(cs[j], inneri, ix[j]))
                vs, ix = nvs, nis
            ps = [jnp.exp(v - vs[0]) for v in vs]
            tot = ps[0]
            for j in range(1, _TOPK):
                tot = tot + ps[j]
            inv = 1.0 / tot
            for j in range(_TOPK):
                kl_v[j, pl.ds(t0, _L)] = vs[j]
                kp_v[j, pl.ds(t0, _L)] = ps[j] * inv
                ki_v[j, pl.ds(t0, _L)] = ix[j]

        def group(g, carry):
            t0 = g * (2 * _L)
            one_group(t0)
            one_group(t0 + _L)
            return carry

        lax.fori_loop(0, rw // (2 * _L), group, 0)

        pltpu.sync_copy(kl_v, kl_hbm.at[:, pl.ds(base, rw)])
        pltpu.sync_copy(kp_v, kp_hbm.at[:, pl.ds(base, rw)])
        pltpu.sync_copy(ki_v, ki_hbm.at[:, pl.ds(base, rw)])

    return _sc_topk_body


def _sc_topk(nchunk):
    rw = nchunk // _NW
    return pl.kernel(
        _make_sc_body(rw),
        out_type=[
            jax.ShapeDtypeStruct((_TOPK, nchunk), jnp.float32),
            jax.ShapeDtypeStruct((_TOPK, nchunk), jnp.float32),
            jax.ShapeDtypeStruct((_TOPK, nchunk), jnp.int32),
        ],
        mesh=plsc.VectorSubcoreMesh(
            core_axis_name="c", subcore_axis_name="s",
            num_cores=_NC, num_subcores=_NS,
        ),
        scratch_types=[
            pltpu.VMEM((_E, rw), jnp.float32),
            pltpu.VMEM((_TOPK, rw), jnp.float32),
            pltpu.VMEM((_TOPK, rw), jnp.float32),
            pltpu.VMEM((_TOPK, rw), jnp.int32),
        ],
    )


def kernel(input, W1, b1, W2, b2):
    b1r = b1.reshape(1, _H)
    b2r = b2.reshape(1, _E)
    logits = None
    kl_parts, kp_parts, ki_parts = [], [], []
    block0 = 0
    for size in _SIZES:
        logits, logits_t_c = _tc_logits_chunk(
            block0, size, input, W1, b1r, W2, b2r, logits)
        kl_t, kp_t, ki_t = _sc_topk(size)(logits_t_c)
        kl_parts.append(kl_t)
        kp_parts.append(kp_t)
        ki_parts.append(ki_t)
        block0 += size // _BN
    kl = jnp.concatenate(kl_parts, axis=1).T
    kp = jnp.concatenate(kp_parts, axis=1).T
    ki = jnp.concatenate(ki_parts, axis=1).T
    return (logits, kl, kp, ki)


# speedup vs baseline: 1.4739x; 1.4739x over previous
import functools

import jax
import jax.numpy as jnp
from jax import lax
from jax.experimental import pallas as pl
from jax.experimental.pallas import tpu as pltpu
from jax.experimental.pallas import tpu_sc as plsc

_N, _D, _H, _E, _TOPK = 32768, 768, 128, 64, 8
_BN = 1024
_SIZES = (8192, 8192, 8192, 8192)

_NC, _NS, _L = 2, 16, 16
_NW = _NC * _NS


def _logits_body_first(x_ref, w1_ref, b1_ref, w2_ref, b2_ref,
                       logits_ref, logits_t_ref):
    h = jnp.dot(x_ref[...], w1_ref[...], preferred_element_type=jnp.float32)
    h = h + b1_ref[...]
    h = h * jax.nn.sigmoid(h)
    w2 = w2_ref[...]
    b2 = b2_ref[...]
    logits = jnp.dot(h, w2, preferred_element_type=jnp.float32)
    logits_ref[...] = logits + b2
    lt = lax.dot_general(w2, h, (((0,), (1,)), ((), ())),
                         preferred_element_type=jnp.float32)
    logits_t_ref[...] = lt + b2.reshape(_E, 1)


def _logits_body_rest(prev_ref, x_ref, w1_ref, b1_ref, w2_ref, b2_ref,
                      logits_ref, logits_t_ref):
    del prev_ref
    _logits_body_first(x_ref, w1_ref, b1_ref, w2_ref, b2_ref,
                       logits_ref, logits_t_ref)


def _tc_logits_chunk(block0, nchunk, x, W1, b1r, W2, b2r, logits_prev):
    gc = nchunk // _BN
    x_spec = pl.BlockSpec((_BN, _D), lambda i: (block0 + i, 0))
    w_specs = [
        pl.BlockSpec((_D, _H), lambda i: (0, 0)),
        pl.BlockSpec((1, _H), lambda i: (0, 0)),
        pl.BlockSpec((_H, _E), lambda i: (0, 0)),
        pl.BlockSpec((1, _E), lambda i: (0, 0)),
    ]
    out_specs = [
        pl.BlockSpec((_BN, _E), lambda i: (block0 + i, 0)),
        pl.BlockSpec((_E, _BN), lambda i: (0, i)),
    ]
    out_shape = [
        jax.ShapeDtypeStruct((_N, _E), jnp.float32),
        jax.ShapeDtypeStruct((_E, nchunk), jnp.float32),
    ]
    if logits_prev is None:
        return pl.pallas_call(
            _logits_body_first,
            grid=(gc,),
            in_specs=[x_spec] + w_specs,
            out_specs=out_specs,
            out_shape=out_shape,
        )(x, W1, b1r, W2, b2r)
    prev_spec = pl.BlockSpec((_BN, _E), lambda i: (block0 + i, 0))
    return pl.pallas_call(
        _logits_body_rest,
        grid=(gc,),
        in_specs=[prev_spec, x_spec] + w_specs,
        out_specs=out_specs,
        out_shape=out_shape,
        input_output_aliases={0: 0},
    )(logits_prev, x, W1, b1r, W2, b2r)


def _make_sc_body(rw):
    def _sc_topk_body(lt_hbm, kl_hbm, kp_hbm, ki_hbm, lg_v, kl_v, kp_v, ki_v):
        wid = lax.axis_index("s") * _NC + lax.axis_index("c")
        base = wid * rw
        pltpu.sync_copy(lt_hbm.at[:, pl.ds(base, rw)], lg_v)

        neg_inf = jnp.full((_L,), -jnp.inf, jnp.float32)
        zero_i = jnp.zeros((_L,), jnp.int32)

        def group(g, carry):
            t0 = g * _L
            vs = [neg_inf] * _TOPK
            ix = [zero_i] * _TOPK
            for e in range(_E):
                nv = lg_v[e, pl.ds(t0, _L)]
                ne = jnp.full((_L,), e, jnp.int32)
                cs = [nv > vs[j] for j in range(_TOPK)]
                nvs, nis = [], []
                for j in range(_TOPK):
                    if j == 0:
                        nvs.append(jnp.where(cs[0], nv, vs[0]))
                        nis.append(jnp.where(cs[0], ne, ix[0]))
                    else:
                        innerv = jnp.where(cs[j - 1], vs[j - 1], nv)
                        inneri = jnp.where(cs[j - 1], ix[j - 1], ne)
                        nvs.append(jnp.where(cs[j], innerv, vs[j]))
                        nis.append(jnp.where(cs[j], inneri, ix[j]))
                vs, ix = nvs, nis
            ps = [jnp.exp(v - vs[0]) for v in vs]
            tot = ps[0]
            for j in range(1, _TOPK):
                tot = tot + ps[j]
            inv = 1.0 / tot
            for j in range(_TOPK):
                kl_v[j, pl.ds(t0, _L)] = vs[j]
                kp_v[j, pl.ds(t0, _L)] = ps[j] * inv
                ki_v[j, pl.ds(t0, _L)] = ix[j]
            return carry

        lax.fori_loop(0, rw // _L, group, 0)

        pltpu.sync_copy(kl_v, kl_hbm.at[:, pl.ds(base, rw)])
        pltpu.sync_copy(kp_v, kp_hbm.at[:, pl.ds(base, rw)])
        pltpu.sync_copy(ki_v, ki_hbm.at[:, pl.ds(base, rw)])

    return _sc_topk_body


def _sc_topk(nchunk):
    rw = nchunk // _NW
    return pl.kernel(
        _make_sc_body(rw),
        out_type=[
            jax.ShapeDtypeStruct((_TOPK, nchunk), jnp.float32),
            jax.ShapeDtypeStruct((_TOPK, nchunk), jnp.float32),
            jax.ShapeDtypeStruct((_TOPK, nchunk), jnp.int32),
        ],
        mesh=plsc.VectorSubcoreMesh(
            core_axis_name="c", subcore_axis_name="s",
            num_cores=_NC, num_subcores=_NS,
        ),
        scratch_types=[
            pltpu.VMEM((_E, rw), jnp.float32),
            pltpu.VMEM((_TOPK, rw), jnp.float32),
            pltpu.VMEM((_TOPK, rw), jnp.float32),
            pltpu.VMEM((_TOPK, rw), jnp.int32),
        ],
    )


def kernel(input, W1, b1, W2, b2):
    b1r = b1.reshape(1, _H)
    b2r = b2.reshape(1, _E)
    logits = None
    kl_parts, kp_parts, ki_parts = [], [], []
    block0 = 0
    for size in _SIZES:
        logits, logits_t_c = _tc_logits_chunk(
            block0, size, input, W1, b1r, W2, b2r, logits)
        kl_t, kp_t, ki_t = _sc_topk(size)(logits_t_c)
        kl_parts.append(kl_t)
        kp_parts.append(kp_t)
        ki_parts.append(ki_t)
        block0 += size // _BN
    kl = jnp.concatenate(kl_parts, axis=1).T
    kp = jnp.concatenate(kp_parts, axis=1).T
    ki = jnp.concatenate(ki_parts, axis=1).T
    return (logits, kl, kp, ki)


# speedup vs baseline: 1.4841x; 1.0069x over previous
import functools

import jax
import jax.numpy as jnp
from jax import lax
from jax.experimental import pallas as pl
from jax.experimental.pallas import tpu as pltpu
from jax.experimental.pallas import tpu_sc as plsc

_N, _D, _H, _E, _TOPK = 32768, 768, 128, 64, 8
_BN = 1024
_SIZES = (8192, 8192, 8192, 8192)

_NC, _NS, _L = 2, 16, 16
_NW = _NC * _NS


def _logits_body_first(x_ref, w1_ref, b1_ref, w2_ref, b2_ref,
                       logits_ref, logits_t_ref):
    h = jnp.dot(x_ref[...], w1_ref[...], preferred_element_type=jnp.float32)
    h = h + b1_ref[...]
    h = h * jax.nn.sigmoid(h)
    w2 = w2_ref[...]
    b2 = b2_ref[...]
    logits = jnp.dot(h, w2, preferred_element_type=jnp.float32)
    logits_ref[...] = logits + b2
    lt = lax.dot_general(w2, h, (((0,), (1,)), ((), ())),
                         preferred_element_type=jnp.float32)
    logits_t_ref[...] = lt + b2.reshape(_E, 1)


def _logits_body_rest(prev_ref, x_ref, w1_ref, b1_ref, w2_ref, b2_ref,
                      logits_ref, logits_t_ref):
    del prev_ref
    _logits_body_first(x_ref, w1_ref, b1_ref, w2_ref, b2_ref,
                       logits_ref, logits_t_ref)


def _tc_logits_chunk(block0, nchunk, x, W1, b1r, W2, b2r, logits_prev):
    gc = nchunk // _BN
    x_spec = pl.BlockSpec((_BN, _D), lambda i: (block0 + i, 0))
    w_specs = [
        pl.BlockSpec((_D, _H), lambda i: (0, 0)),
        pl.BlockSpec((1, _H), lambda i: (0, 0)),
        pl.BlockSpec((_H, _E), lambda i: (0, 0)),
        pl.BlockSpec((1, _E), lambda i: (0, 0)),
    ]
    out_specs = [
        pl.BlockSpec((_BN, _E), lambda i: (block0 + i, 0)),
        pl.BlockSpec((_E, _BN), lambda i: (0, i)),
    ]
    out_shape = [
        jax.ShapeDtypeStruct((_N, _E), jnp.float32),
        jax.ShapeDtypeStruct((_E, nchunk), jnp.float32),
    ]
    if logits_prev is None:
        return pl.pallas_call(
            _logits_body_first,
            grid=(gc,),
            in_specs=[x_spec] + w_specs,
            out_specs=out_specs,
            out_shape=out_shape,
        )(x, W1, b1r, W2, b2r)
    prev_spec = pl.BlockSpec((_BN, _E), lambda i: (block0 + i, 0))
    return pl.pallas_call(
        _logits_body_rest,
        grid=(gc,),
        in_specs=[prev_spec, x_spec] + w_specs,
        out_specs=out_specs,
        out_shape=out_shape,
        input_output_aliases={0: 0},
    )(logits_prev, x, W1, b1r, W2, b2r)


def _make_sc_body(rw):
    def _sc_topk_body(lt_hbm, out_hbm, lg_v, out_v):
        wid = lax.axis_index("s") * _NC + lax.axis_index("c")
        base = wid * rw
        pltpu.sync_copy(lt_hbm.at[:, pl.ds(base, rw)], lg_v)

        neg_inf = jnp.full((_L,), -jnp.inf, jnp.float32)
        zero_i = jnp.zeros((_L,), jnp.int32)

        def group(g, carry):
            t0 = g * _L
            vs = [neg_inf] * _TOPK
            ix = [zero_i] * _TOPK
            for e in range(_E):
                nv = lg_v[e, pl.ds(t0, _L)]
                ne = jnp.full((_L,), e, jnp.int32)
                cs = [nv > vs[j] for j in range(_TOPK)]
                nvs, nis = [], []
                for j in range(_TOPK):
                    if j == 0:
                        nvs.append(jnp.where(cs[0], nv, vs[0]))
                        nis.append(jnp.where(cs[0], ne, ix[0]))
                    else:
                        innerv = jnp.where(cs[j - 1], vs[j - 1], nv)
                        inneri = jnp.where(cs[j - 1], ix[j - 1], ne)
                        nvs.append(jnp.where(cs[j], innerv, vs[j]))
                        nis.append(jnp.where(cs[j], inneri, ix[j]))
                vs, ix = nvs, nis
            ps = [jnp.exp(v - vs[0]) for v in vs]
            tot = ps[0]
            for j in range(1, _TOPK):
                tot = tot + ps[j]
            inv = 1.0 / tot
            for j in range(_TOPK):
                out_v[j, pl.ds(t0, _L)] = vs[j]
                out_v[_TOPK + j, pl.ds(t0, _L)] = ps[j] * inv
                out_v[2 * _TOPK + j, pl.ds(t0, _L)] = (
                    lax.bitcast_convert_type(ix[j], jnp.float32))
            return carry

        lax.fori_loop(0, rw // _L, group, 0)

        pltpu.sync_copy(out_v, out_hbm.at[:, pl.ds(base, rw)])

    return _sc_topk_body


def _sc_topk(nchunk):
    rw = nchunk // _NW
    return pl.kernel(
        _make_sc_body(rw),
        out_type=jax.ShapeDtypeStruct((3 * _TOPK, nchunk), jnp.float32),
        mesh=plsc.VectorSubcoreMesh(
            core_axis_name="c", subcore_axis_name="s",
            num_cores=_NC, num_subcores=_NS,
        ),
        scratch_types=[
            pltpu.VMEM((_E, rw), jnp.float32),
            pltpu.VMEM((3 * _TOPK, rw), jnp.float32),
        ],
    )


def kernel(input, W1, b1, W2, b2):
    b1r = b1.reshape(1, _H)
    b2r = b2.reshape(1, _E)
    logits = None
    parts = []
    block0 = 0
    for size in _SIZES:
        logits, logits_t_c = _tc_logits_chunk(
            block0, size, input, W1, b1r, W2, b2r, logits)
        parts.append(_sc_topk(size)(logits_t_c))
        block0 += size // _BN
    packed = jnp.concatenate(parts, axis=1)
    kl = packed[:_TOPK].T
    kp = packed[_TOPK:2 * _TOPK].T
    ki = lax.bitcast_convert_type(packed[2 * _TOPK:].T, jnp.int32)
    return (logits, kl, kp, ki)
